# baseline (speedup 1.0000x reference)
import functools

import jax
import jax.numpy as jnp
from jax import lax
from jax.experimental import pallas as pl
from jax.experimental.pallas import tpu as pltpu
from jax.experimental.pallas import tpu_sc as plsc

_NC = 2
_NS = 16
_NW = _NC * _NS
_CH = 128
_BN = 16384
_SENT = -1


def _mm_body(tt_ref, wt_ref, o_ref):
    o_ref[...] = lax.dot_general(
        tt_ref[...], wt_ref[...],
        dimension_numbers=(((0,), (0,)), ((), ())),
        preferred_element_type=jnp.float32,
    )


def _tc_project(tt, wt, block_lo, n_blocks):
    hid = tt.shape[0]
    emb = wt.shape[1]
    return pl.pallas_call(
        _mm_body,
        grid=(n_blocks,),
        in_specs=[
            pl.BlockSpec((hid, _BN), lambda i: (0, i + block_lo)),
            pl.BlockSpec((hid, emb), lambda i: (0, 0)),
        ],
        out_specs=pl.BlockSpec((_BN, emb), lambda i: (i, 0)),
        out_shape=jax.ShapeDtypeStruct((n_blocks * _BN, emb), jnp.float32),
    )(tt, wt)


def _sc_gather_pass(proj, idx3, out_ref, lo, hi, scatter_out):
    n_chunks = idx3.shape[1]
    emb = proj.shape[1]

    @functools.partial(
        pl.kernel,
        mesh=plsc.VectorSubcoreMesh(core_axis_name="c", subcore_axis_name="s"),
        out_type=(),
        scratch_types=[
            pltpu.VMEM((n_chunks, _CH), jnp.int32),
            *[pltpu.VMEM((_CH, emb), jnp.float32) for _ in range(4)],
            *[pltpu.VMEM((_CH,), jnp.int32) for _ in range(4)],
            *[pltpu.VMEM((_CH,), jnp.int32) for _ in range(4)],
            *[pltpu.SemaphoreType.DMA for _ in range(8)],
        ],
    )
    def k(proj_hbm, idx_hbm, out_hbm, idx_v,
          r0, r1, r2, r3, g0, g1, g2, g3, s0, s1, s2, s3,
          sg0, sg1, sg2, sg3, ss0, ss1, ss2, ss3):
        rows = (r0, r1, r2, r3)
        gbufs = (g0, g1, g2, g3)
        sbufs = (s0, s1, s2, s3)
        sgs = (sg0, sg1, sg2, sg3)
        sss = (ss0, ss1, ss2, ss3)
        wid = lax.axis_index("s") * _NC + lax.axis_index("c")
        base = wid * (n_chunks * _CH)
        pltpu.sync_copy(idx_hbm.at[wid], idx_v)

        def prep(c, b):
            bc = base + c * _CH
            for j in range(_CH // 16):
                v = idx_v[c, pl.ds(j * 16, 16)]
                if lo == 0:
                    m = v < hi
                    g = jnp.where(m, v, _SENT)
                else:
                    m = v >= lo
                    g = jnp.where(m, v - lo, _SENT)
                gbufs[b][pl.ds(j * 16, 16)] = g
                if scatter_out:
                    s = jnp.where(
                        m, bc + j * 16 + lax.iota(jnp.int32, 16), _SENT)
                    sbufs[b][pl.ds(j * 16, 16)] = s

        def g_copy(c, b):
            del c
            return pltpu.make_async_copy(
                proj_hbm.at[plsc.Indices(gbufs[b], ignored_value=_SENT)],
                rows[b], sgs[b])

        def s_copy(c, b):
            if scatter_out:
                return pltpu.make_async_copy(
                    rows[b],
                    out_hbm.at[plsc.Indices(sbufs[b], ignored_value=_SENT)],
                    sss[b])
            return pltpu.make_async_copy(
                rows[b], out_hbm.at[pl.ds(base + c * _CH, _CH)], sss[b])

        prep(0, 0)
        g_copy(0, 0).start()
        prep(1, 1)
        g_copy(1, 1).start()

        def body(i, carry):
            for k4 in range(4):
                c = i * 4 + k4
                b = k4
                b2 = (k4 + 2) % 4
                g_copy(c, b).wait()
                s_copy(c, b).start()

                @pl.when(c >= 2)
                def _():
                    s_copy(c - 2, b2).wait()

                @pl.when(c + 2 < n_chunks)
                def _():
                    prep(c + 2, b2)
                    g_copy(c + 2, b2).start()
            return carry

        lax.fori_loop(0, n_chunks // 4, body, 0)
        s_copy(n_chunks - 2, (n_chunks - 2) % 4).wait()
        s_copy(n_chunks - 1, (n_chunks - 1) % 4).wait()

    k(proj, idx3, out_ref)


def kernel(x, emb_table, W_up):
    b, l = x.shape
    t_total = b * l
    v, hid = emb_table.shape
    emb = W_up.shape[0]
    tt = emb_table.T
    wt = W_up.T

    n_blocks = pl.cdiv(v, _BN)
    nb1 = n_blocks // 2
    v_lo = nb1 * _BN

    proj1 = _tc_project(tt, wt, 0, nb1)
    proj2 = _tc_project(tt, wt, nb1, n_blocks - nb1)

    idx3 = x.reshape(_NW, t_total // (_NW * _CH), _CH).astype(jnp.int32)
    out_ref = jax.empty_ref(
        jax.ShapeDtypeStruct((t_total, emb), jnp.float32))
    _sc_gather_pass(proj1, idx3, out_ref, 0, v_lo, scatter_out=False)
    _sc_gather_pass(proj2, idx3, out_ref, v_lo, v, scatter_out=True)
    return out_ref[...].reshape(b, l, emb)


# speedup vs baseline: 1.2118x; 1.2118x over previous
import functools

import jax
import jax.numpy as jnp
from jax import lax
from jax.experimental import pallas as pl
from jax.experimental.pallas import tpu as pltpu
from jax.experimental.pallas import tpu_sc as plsc

_NC = 2
_NS = 16
_NW = _NC * _NS
_CH = 128
_BN = 16384


def _mm_body(tt_ref, wt_ref, o_ref):
    o_ref[...] = lax.dot_general(
        tt_ref[...], wt_ref[...],
        dimension_numbers=(((0,), (0,)), ((), ())),
        preferred_element_type=jnp.float32,
    )


def _tc_project(tt, wt):
    hid, v = tt.shape
    emb = wt.shape[1]
    n_blocks = pl.cdiv(v, _BN)
    return pl.pallas_call(
        _mm_body,
        grid=(n_blocks,),
        in_specs=[
            pl.BlockSpec((hid, _BN), lambda i: (0, i)),
            pl.BlockSpec((hid, emb), lambda i: (0, 0)),
        ],
        out_specs=pl.BlockSpec((_BN, emb), lambda i: (i, 0)),
        out_shape=jax.ShapeDtypeStruct((n_blocks * _BN, emb), jnp.float32),
    )(tt, wt)


def _sc_gather(proj, idx3):
    n_chunks = idx3.shape[1]
    emb = proj.shape[1]
    t_total = _NW * n_chunks * _CH

    @functools.partial(
        pl.kernel,
        mesh=plsc.VectorSubcoreMesh(core_axis_name="c", subcore_axis_name="s"),
        out_type=jax.ShapeDtypeStruct((t_total, emb), jnp.float32),
        scratch_types=[
            pltpu.VMEM((n_chunks, _CH), jnp.int32),
            *[pltpu.VMEM((_CH, emb), jnp.float32) for _ in range(4)],
            *[pltpu.SemaphoreType.DMA for _ in range(8)],
        ],
    )
    def k(proj_hbm, idx_hbm, out_hbm, idx_v, r0, r1, r2, r3,
          sg0, sg1, sg2, sg3, ss0, ss1, ss2, ss3):
        rows = (r0, r1, r2, r3)
        sgs = (sg0, sg1, sg2, sg3)
        sss = (ss0, ss1, ss2, ss3)
        wid = lax.axis_index("s") * _NC + lax.axis_index("c")
        base = wid * (n_chunks * _CH)
        pltpu.sync_copy(idx_hbm.at[wid], idx_v)

        def g_copy(c, b):
            return pltpu.make_async_copy(
                proj_hbm.at[idx_v.at[c]], rows[b], sgs[b])

        def s_copy(c, b):
            return pltpu.make_async_copy(
                rows[b], out_hbm.at[pl.ds(base + c * _CH, _CH)], sss[b])

        g_copy(0, 0).start()
        g_copy(1, 1).start()

        def body(i, carry):
            for k4 in range(4):
                c = i * 4 + k4
                b = k4
                b2 = (k4 + 2) % 4
                g_copy(c, b).wait()
                s_copy(c, b).start()

                @pl.when(c >= 2)
                def _():
                    s_copy(c - 2, b2).wait()

                @pl.when(c + 2 < n_chunks)
                def _():
                    g_copy(c + 2, b2).start()
            return carry

        lax.fori_loop(0, n_chunks // 4, body, 0)
        s_copy(n_chunks - 2, (n_chunks - 2) % 4).wait()
        s_copy(n_chunks - 1, (n_chunks - 1) % 4).wait()

    return k(proj, idx3)


def kernel(x, emb_table, W_up):
    b, l = x.shape
    t_total = b * l
    emb = W_up.shape[0]
    proj = _tc_project(emb_table.T, W_up.T)
    idx3 = x.reshape(_NW, t_total // (_NW * _CH), _CH).astype(jnp.int32)
    out = _sc_gather(proj, idx3)
    return out.reshape(b, l, emb)


# speedup vs baseline: 1.2268x; 1.0123x over previous
import functools

import jax
import jax.numpy as jnp
from jax import lax
from jax.experimental import pallas as pl
from jax.experimental.pallas import tpu as pltpu
from jax.experimental.pallas import tpu_sc as plsc

_NC = 2
_NS = 16
_NW = _NC * _NS
_CH = 128
_BN = 32768


def _mm_body(tt_ref, wt_ref, o_ref):
    o_ref[...] = lax.dot_general(
        tt_ref[...], wt_ref[...],
        dimension_numbers=(((0,), (0,)), ((), ())),
        preferred_element_type=jnp.float32,
    )


def _tc_project(tt, wt):
    hid, v = tt.shape
    emb = wt.shape[1]
    n_blocks = pl.cdiv(v, _BN)
    return pl.pallas_call(
        _mm_body,
        grid=(n_blocks,),
        in_specs=[
            pl.BlockSpec((hid, _BN), lambda i: (0, i)),
            pl.BlockSpec((hid, emb), lambda i: (0, 0)),
        ],
        out_specs=pl.BlockSpec((_BN, emb), lambda i: (i, 0)),
        out_shape=jax.ShapeDtypeStruct((n_blocks * _BN, emb), jnp.float32),
    )(tt, wt)


def _sc_gather(proj, idx3):
    n_chunks = idx3.shape[1]
    emb = proj.shape[1]
    t_total = _NW * n_chunks * _CH

    @functools.partial(
        pl.kernel,
        mesh=plsc.VectorSubcoreMesh(core_axis_name="c", subcore_axis_name="s"),
        out_type=jax.ShapeDtypeStruct((t_total, emb), jnp.float32),
        scratch_types=[
            pltpu.VMEM((n_chunks, _CH), jnp.int32),
            *[pltpu.VMEM((_CH, emb), jnp.float32) for _ in range(4)],
            *[pltpu.SemaphoreType.DMA for _ in range(8)],
        ],
    )
    def k(proj_hbm, idx_hbm, out_hbm, idx_v, r0, r1, r2, r3,
          sg0, sg1, sg2, sg3, ss0, ss1, ss2, ss3):
        rows = (r0, r1, r2, r3)
        sgs = (sg0, sg1, sg2, sg3)
        sss = (ss0, ss1, ss2, ss3)
        wid = lax.axis_index("s") * _NC + lax.axis_index("c")
        base = wid * (n_chunks * _CH)
        pltpu.sync_copy(idx_hbm.at[wid], idx_v)

        def g_copy(c, b):
            return pltpu.make_async_copy(
                proj_hbm.at[idx_v.at[c]], rows[b], sgs[b])

        def s_copy(c, b):
            return pltpu.make_async_copy(
                rows[b], out_hbm.at[pl.ds(base + c * _CH, _CH)], sss[b])

        g_copy(0, 0).start()
        g_copy(1, 1).start()

        def body(i, carry):
            for k4 in range(4):
                c = i * 4 + k4
                b = k4
                b2 = (k4 + 2) % 4
                g_copy(c, b).wait()
                s_copy(c, b).start()

                @pl.when(c >= 2)
                def _():
                    s_copy(c - 2, b2).wait()

                @pl.when(c + 2 < n_chunks)
                def _():
                    g_copy(c + 2, b2).start()
            return carry

        lax.fori_loop(0, n_chunks // 4, body, 0)
        s_copy(n_chunks - 2, (n_chunks - 2) % 4).wait()
        s_copy(n_chunks - 1, (n_chunks - 1) % 4).wait()

    return k(proj, idx3)


def kernel(x, emb_table, W_up):
    b, l = x.shape
    t_total = b * l
    emb = W_up.shape[0]
    proj = _tc_project(emb_table.T, W_up.T)
    idx3 = x.reshape(_NW, t_total // (_NW * _CH), _CH).astype(jnp.int32)
    out = _sc_gather(proj, idx3)
    return out.reshape(b, l, emb)
